# Initial kernel scaffold; baseline (speedup 1.0000x reference)
import jax
import jax.numpy as jnp
from jax.experimental import pallas as pl


def kernel(x, edge_index, edge_label_index, W1_l, b1_l, W1_r, W2_l, b2_l, W2_r):
    raise NotImplementedError("write your pallas kernel here")


# speedup vs baseline: 4.5517x; 4.5517x over previous
import jax
import jax.numpy as jnp
from jax import lax
from jax.experimental import pallas as pl
from jax.experimental.pallas import tpu as pltpu
from jax.experimental.pallas import tpu_sc as plsc

N_NODES = 10000
D = 128
E = 320000
L_LAB = 200000

NC = 2
NS = 16
NW = NC * NS
CHUNK = 128
N_CHUNKS = E // CHUNK
N_PAD = 10240
ROWS_PER_TILE = N_PAD // NS
ZCOPY = 5
ZROWS = ROWS_PER_TILE // ZCOPY
LAB_CPT = 49
LAB_PAD = NW * LAB_CPT * CHUNK

_MESH = plsc.VectorSubcoreMesh(core_axis_name="c", subcore_axis_name="s",
                               num_cores=NC, num_subcores=NS)


def _agg_body(src_hbm, dst_hbm, table_hbm, out_acc, acc_sh, sidx, didx,
              rows, sem):
  cid = lax.axis_index("c")
  sid = lax.axis_index("s")
  wid = cid * NS + sid

  def zrow(i, _):
    for j in range(D // 16):
      rows[i, pl.ds(j * 16, 16)] = jnp.zeros((16,), jnp.float32)
    return 0
  lax.fori_loop(0, ZROWS, zrow, 0)
  r0 = sid * ROWS_PER_TILE
  for k in range(ZCOPY):
    pltpu.sync_copy(rows.at[pl.ds(0, ZROWS)],
                    acc_sh.at[pl.ds(r0 + k * ZROWS, ZROWS)])
  plsc.subcore_barrier()

  c_lo = (wid * N_CHUNKS) // NW
  c_hi = ((wid + 1) * N_CHUNKS) // NW

  def step(c, _):
    pltpu.sync_copy(src_hbm.at[c], sidx)
    pltpu.sync_copy(dst_hbm.at[c], didx)
    pltpu.async_copy(table_hbm.at[sidx], rows, sem).wait()
    pltpu.sync_copy(rows, acc_sh.at[didx], add=True)
    return 0
  lax.fori_loop(c_lo, c_hi, step, 0)

  plsc.subcore_barrier()
  for k in range(ZCOPY):
    rs = r0 + k * ZROWS
    pltpu.sync_copy(acc_sh.at[pl.ds(rs, ZROWS)],
                    out_acc.at[cid, pl.ds(rs, ZROWS)])


_agg = pl.kernel(
    _agg_body,
    out_type=jax.ShapeDtypeStruct((NC, N_PAD, D), jnp.float32),
    mesh=_MESH,
    scratch_types=(
        pltpu.VMEM_SHARED((N_PAD, D), jnp.float32),
        pltpu.VMEM((CHUNK,), jnp.int32),
        pltpu.VMEM((CHUNK,), jnp.int32),
        pltpu.VMEM((CHUNK, D), jnp.float32),
        pltpu.SemaphoreType.DMA,
    ),
)


def _count_body(dst_hbm, out_cnt, cnt_sh, didx, ones):
  cid = lax.axis_index("c")
  sid = lax.axis_index("s")
  wid = cid * NS + sid

  def zrow(i, _):
    for j in range(D // 16):
      ones[i, pl.ds(j * 16, 16)] = jnp.zeros((16,), jnp.float32)
    return 0
  lax.fori_loop(0, ZROWS, zrow, 0)
  r0 = sid * ROWS_PER_TILE
  for k in range(ZCOPY):
    pltpu.sync_copy(ones.at[pl.ds(0, ZROWS)],
                    cnt_sh.at[pl.ds(r0 + k * ZROWS, ZROWS)])
  def orow(i, _):
    for j in range(D // 16):
      ones[i, pl.ds(j * 16, 16)] = jnp.ones((16,), jnp.float32)
    return 0
  lax.fori_loop(0, ZROWS, orow, 0)
  plsc.subcore_barrier()

  c_lo = (wid * N_CHUNKS) // NW
  c_hi = ((wid + 1) * N_CHUNKS) // NW

  def step(c, _):
    pltpu.sync_copy(dst_hbm.at[c], didx)
    pltpu.sync_copy(ones, cnt_sh.at[didx], add=True)
    return 0
  lax.fori_loop(c_lo, c_hi, step, 0)

  plsc.subcore_barrier()
  for k in range(ZCOPY):
    rs = r0 + k * ZROWS
    pltpu.sync_copy(cnt_sh.at[pl.ds(rs, ZROWS)],
                    out_cnt.at[cid, pl.ds(rs, ZROWS)])


_count = pl.kernel(
    _count_body,
    out_type=jax.ShapeDtypeStruct((NC, N_PAD, D), jnp.float32),
    mesh=_MESH,
    scratch_types=(
        pltpu.VMEM_SHARED((N_PAD, D), jnp.float32),
        pltpu.VMEM((CHUNK,), jnp.int32),
        pltpu.VMEM((CHUNK, D), jnp.float32),
    ),
)


_B = 400


def _make_dense(with_relu):
  def body(acc_ref, inv_ref, x_ref, wl_ref, bl_ref, wr_ref, o_ref):
    agg = acc_ref[0] + acc_ref[1]
    mean = agg * inv_ref[...]
    out = (lax.dot_general(mean, wl_ref[...], (((1,), (1,)), ((), ())),
                           preferred_element_type=jnp.float32)
           + bl_ref[...]
           + lax.dot_general(x_ref[...], wr_ref[...], (((1,), (1,)), ((), ())),
                             preferred_element_type=jnp.float32))
    o_ref[...] = jnp.maximum(out, 0.0) if with_relu else out

  return pl.pallas_call(
      body,
      grid=(N_NODES // _B,),
      in_specs=[
          pl.BlockSpec((NC, _B, D), lambda i: (0, i, 0)),
          pl.BlockSpec((_B, 1), lambda i: (i, 0)),
          pl.BlockSpec((_B, D), lambda i: (i, 0)),
          pl.BlockSpec((D, D), lambda i: (0, 0)),
          pl.BlockSpec((1, D), lambda i: (0, 0)),
          pl.BlockSpec((D, D), lambda i: (0, 0)),
      ],
      out_specs=pl.BlockSpec((_B, D), lambda i: (i, 0)),
      out_shape=jax.ShapeDtypeStruct((N_NODES, D), jnp.float32),
  )


_dense_relu = _make_dense(True)
_dense_plain = _make_dense(False)


def _decode_body(ls_hbm, ld_hbm, z_hbm, out_hbm, sidx, didx, ra, rb, pbuf, sem):
  cid = lax.axis_index("c")
  sid = lax.axis_index("s")
  wid = cid * NS + sid

  def chunk_step(k, _):
    c = wid * LAB_CPT + k
    pltpu.sync_copy(ls_hbm.at[c], sidx)
    pltpu.sync_copy(ld_hbm.at[c], didx)
    cp_a = pltpu.async_copy(z_hbm.at[sidx], ra, sem)
    cp_b = pltpu.async_copy(z_hbm.at[didx], rb, sem)
    cp_a.wait()
    cp_b.wait()

    def pair_step(p, _):
      acc = jnp.zeros((16,), jnp.float32)
      for j in range(D // 16):
        acc = acc + ra[p, pl.ds(j * 16, 16)] * rb[p, pl.ds(j * 16, 16)]
      pbuf[p] = acc
      return 0
    lax.fori_loop(0, CHUNK, pair_step, 0)
    pltpu.sync_copy(pbuf, out_hbm.at[pl.ds(c * CHUNK, CHUNK)])
    return 0
  lax.fori_loop(0, LAB_CPT, chunk_step, 0)


_decode = pl.kernel(
    _decode_body,
    out_type=jax.ShapeDtypeStruct((LAB_PAD, 16), jnp.float32),
    mesh=_MESH,
    scratch_types=(
        pltpu.VMEM((CHUNK,), jnp.int32),
        pltpu.VMEM((CHUNK,), jnp.int32),
        pltpu.VMEM((CHUNK, D), jnp.float32),
        pltpu.VMEM((CHUNK, D), jnp.float32),
        pltpu.VMEM((CHUNK, 16), jnp.float32),
        pltpu.SemaphoreType.DMA,
    ),
)


def _reduce_body(p_ref, o_ref):
  o_ref[...] = jnp.sum(p_ref[...], axis=-1)


_RB = 4096
_reduce16 = pl.pallas_call(
    _reduce_body,
    grid=(LAB_PAD // _RB,),
    in_specs=[pl.BlockSpec((_RB, 16), lambda i: (i, 0))],
    out_specs=pl.BlockSpec((_RB,), lambda i: (i,)),
    out_shape=jax.ShapeDtypeStruct((LAB_PAD,), jnp.float32),
)


def kernel(x, edge_index, edge_label_index, W1_l, b1_l, W1_r, W2_l, b2_l, W2_r):
  ei = edge_index.astype(jnp.int32)
  src2d = ei[0].reshape(N_CHUNKS, CHUNK)
  dst2d = ei[1].reshape(N_CHUNKS, CHUNK)
  eli = edge_label_index.astype(jnp.int32)
  pad = LAB_PAD - L_LAB
  ls2d = jnp.pad(eli[0], (0, pad)).reshape(LAB_PAD // CHUNK, CHUNK)
  ld2d = jnp.pad(eli[1], (0, pad)).reshape(LAB_PAD // CHUNK, CHUNK)

  cntacc = _count(dst2d)
  cnt = cntacc[0, :, 0] + cntacc[1, :, 0]
  inv = (1.0 / jnp.maximum(cnt, 1.0))[:N_NODES, None]
  acc1 = _agg(src2d, dst2d, x)
  h = _dense_relu(acc1, inv, x, W1_l, b1_l.reshape(1, D), W1_r)
  acc2 = _agg(src2d, dst2d, h)
  z = _dense_plain(acc2, inv, h, W2_l, b2_l.reshape(1, D), W2_r)
  parts = _decode(ls2d, ld2d, z)
  scores = _reduce16(parts)
  return scores[:L_LAB]
